# baseline (speedup 1.0000x reference)
import jax
import jax.numpy as jnp
from jax.experimental import pallas as pl
from jax.experimental.pallas import tpu as pltpu

N_EMBD = 4096
HIDDEN = N_EMBD // 4
N_EXPERTS = 64
TOP_K = 8
BM = 1024


def _router_block(x_ref, w1_ref, w2_ref, ew_ref, mask_ref, probs_ref, lg_ref):
    i = pl.program_id(0)

    logits_prev = lg_ref[(i + 1) % 2]
    mx = jnp.max(logits_prev, axis=0, keepdims=True)
    e = jnp.exp(logits_prev - mx)
    probs = e / jnp.sum(e, axis=0, keepdims=True)

    inv_idx = N_EXPERTS - 1 - jax.lax.broadcasted_iota(jnp.int32, probs.shape, 0)
    work = probs
    mask = jnp.zeros_like(probs)
    for _ in range(TOP_K):
        m = jnp.max(work, axis=0, keepdims=True)
        is_max = work == m
        cand = jnp.where(is_max, inv_idx, -1)
        win = jnp.max(cand, axis=0, keepdims=True)
        sel = cand == win
        mask = mask + sel.astype(jnp.float32)
        work = jnp.where(sel, -1.0, work)

    masked = probs * mask
    wsum = jnp.maximum(jnp.sum(masked, axis=0, keepdims=True), 1e-8)
    ew = masked / wsum

    probs_ref[...] = probs.T
    mask_ref[...] = mask.T
    ew_ref[...] = ew.T

    x = x_ref[...]
    h = jax.lax.dot_general(
        x, w1_ref[...], (((1,), (1,)), ((), ())),
        preferred_element_type=jnp.float32)
    h = h * jax.nn.sigmoid(h)
    logits = jax.lax.dot_general(
        w2_ref[...], h, (((1,), (1,)), ((), ())),
        preferred_element_type=jnp.float32)
    lg_ref[i % 2] = logits


def kernel(x, W1, W2):
    B, T, E = x.shape
    M = B * T
    N = M // BM
    xf = x.reshape(M, E)
    outs = pl.pallas_call(
        _router_block,
        grid=(N + 1,),
        in_specs=[
            pl.BlockSpec((BM, E), lambda i: (jnp.minimum(i, N - 1), 0)),
            pl.BlockSpec((HIDDEN, E), lambda i: (0, 0)),
            pl.BlockSpec((N_EXPERTS, HIDDEN), lambda i: (0, 0)),
        ],
        out_specs=[pl.BlockSpec((BM, N_EXPERTS),
                                lambda i: (jnp.maximum(i - 1, 0), 0))] * 3,
        out_shape=[jax.ShapeDtypeStruct((M, N_EXPERTS), jnp.float32)] * 3,
        scratch_shapes=[pltpu.VMEM((2, N_EXPERTS, BM), jnp.float32)],
    )(xf, W1, W2)
    ew, mask, probs = (o.reshape(B, T, N_EXPERTS) for o in outs)
    return (ew, mask, probs)


# speedup vs baseline: 1.1088x; 1.1088x over previous
import jax
import jax.numpy as jnp
from jax.experimental import pallas as pl

N_EMBD = 4096
HIDDEN = N_EMBD // 4
N_EXPERTS = 64
TOP_K = 8
BM = 512


def _router_block(x_ref, w1_ref, w2_ref, ew_ref, mask_ref, probs_ref):
    x = x_ref[...]
    h = jax.lax.dot_general(
        x, w1_ref[...], (((1,), (1,)), ((), ())),
        preferred_element_type=jnp.float32)
    h = h * jax.nn.sigmoid(h)
    logits = jax.lax.dot_general(
        w2_ref[...], h, (((1,), (1,)), ((), ())),
        preferred_element_type=jnp.float32)

    mx = jnp.max(logits, axis=0, keepdims=True)
    e = jnp.exp(logits - mx)
    probs = e / jnp.sum(e, axis=0, keepdims=True)
    probs_ref[...] = probs

    inv_idx = N_EXPERTS - 1 - jax.lax.broadcasted_iota(jnp.int32, probs.shape, 0)
    work = probs
    mask = jnp.zeros_like(probs)
    for _ in range(TOP_K):
        m = jnp.max(work, axis=0, keepdims=True)
        is_max = work == m
        cand = jnp.where(is_max, inv_idx, -1)
        win = jnp.max(cand, axis=0, keepdims=True)
        sel = cand == win
        mask = mask + sel.astype(jnp.float32)
        work = jnp.where(sel, -1.0, work)
    mask_ref[...] = mask

    masked = probs * mask
    wsum = jnp.maximum(jnp.sum(masked, axis=0, keepdims=True), 1e-8)
    ew_ref[...] = masked / wsum


def kernel(x, W1, W2):
    B, T, E = x.shape
    M = B * T
    xf = x.reshape(M, E)
    outs = pl.pallas_call(
        _router_block,
        grid=(M // BM,),
        in_specs=[
            pl.BlockSpec((BM, E), lambda i: (i, 0)),
            pl.BlockSpec((HIDDEN, E), lambda i: (0, 0)),
            pl.BlockSpec((N_EXPERTS, HIDDEN), lambda i: (0, 0)),
        ],
        out_specs=[pl.BlockSpec((N_EXPERTS, BM), lambda i: (0, i))] * 3,
        out_shape=[jax.ShapeDtypeStruct((N_EXPERTS, M), jnp.float32)] * 3,
    )(xf, W1, W2)
    ew, mask, probs = (o.T.reshape(B, T, N_EXPERTS) for o in outs)
    return (ew, mask, probs)


# speedup vs baseline: 1.1343x; 1.0230x over previous
import jax
import jax.numpy as jnp
from jax.experimental import pallas as pl

N_EMBD = 4096
HIDDEN = N_EMBD // 4
N_EXPERTS = 64
TOP_K = 8
BM = 1024
RCHUNK = 4


def _router_block(x_ref, w1_ref, w2_ref, ew_ref, mask_ref, probs_ref):
    x = x_ref[...]
    h = jax.lax.dot_general(
        x, w1_ref[...], (((1,), (1,)), ((), ())),
        preferred_element_type=jnp.float32)
    h = h * jax.nn.sigmoid(h)
    logits = jax.lax.dot_general(
        w2_ref[...], h, (((1,), (1,)), ((), ())),
        preferred_element_type=jnp.float32)

    cw = BM // RCHUNK
    inv_idx = jnp.int32(N_EXPERTS - 1) - jax.lax.broadcasted_iota(
        jnp.int32, (N_EXPERTS, cw), 0)
    for c in range(RCHUNK):
        cols = slice(c * cw, (c + 1) * cw)
        lg = logits[:, cols]

        mx = jnp.max(lg, axis=0, keepdims=True)
        e = jnp.exp(lg - mx)
        probs = e / jnp.sum(e, axis=0, keepdims=True)
        probs_ref[:, cols] = probs

        work = probs
        mask = jnp.zeros_like(probs)
        for _ in range(TOP_K):
            m = jnp.max(work, axis=0, keepdims=True)
            is_max = work == m
            cand = jnp.where(is_max, inv_idx, -1)
            win = jnp.max(cand, axis=0, keepdims=True)
            sel = cand == win
            mask = mask + sel.astype(jnp.float32)
            work = jnp.where(sel, -1.0, work)
        mask_ref[:, cols] = mask

        masked = probs * mask
        wsum = jnp.maximum(jnp.sum(masked, axis=0, keepdims=True), 1e-8)
        ew_ref[:, cols] = masked / wsum


def kernel(x, W1, W2):
    B, T, E = x.shape
    M = B * T
    xf = x.reshape(M, E)
    outs = pl.pallas_call(
        _router_block,
        grid=(M // BM,),
        in_specs=[
            pl.BlockSpec((BM, E), lambda i: (i, 0)),
            pl.BlockSpec((HIDDEN, E), lambda i: (0, 0)),
            pl.BlockSpec((N_EXPERTS, HIDDEN), lambda i: (0, 0)),
        ],
        out_specs=[pl.BlockSpec((N_EXPERTS, BM), lambda i: (0, i))] * 3,
        out_shape=[jax.ShapeDtypeStruct((N_EXPERTS, M), jnp.float32)] * 3,
    )(xf, W1, W2)
    ew, mask, probs = (o.T.reshape(B, T, N_EXPERTS) for o in outs)
    return (ew, mask, probs)
